# baseline (speedup 1.0000x reference)
import functools

import jax
import jax.numpy as jnp
from jax import lax
from jax.experimental import pallas as pl
from jax.experimental.pallas import tpu as pltpu
from jax.experimental.pallas import tpu_sc as plsc

B = 16384
D = 64
NEG = 5
NEG_PAD = 8

NUM_SC = 2
NUM_SUBCORES = 16
NW = NUM_SC * NUM_SUBCORES
ROWS_PER_W = B // NW
CHUNK = 128
NCHUNK = ROWS_PER_W // CHUNK


def _sc_gather(x_hbm, y_hbm, neg_hbm, wi_hbm, wo_hbm,
               vi_out, vo_out, samp_out,
               xidx_v, yidx_v, negidx_v, xrows_v, yrows_v, negrows_v,
               sem_x, sem_y, sem_n):
    wid = lax.axis_index("s") * NUM_SC + lax.axis_index("c")
    base = wid * ROWS_PER_W

    pltpu.sync_copy(x_hbm.at[pl.ds(base, ROWS_PER_W)], xidx_v)
    pltpu.sync_copy(y_hbm.at[pl.ds(base, ROWS_PER_W)], yidx_v)

    copies = []
    for j in range(NCHUNK):
        sl = pl.ds(j * CHUNK, CHUNK)
        copies.append(pltpu.async_copy(wi_hbm.at[xidx_v.at[sl]],
                                       xrows_v.at[sl], sem_x))
        copies.append(pltpu.async_copy(wo_hbm.at[yidx_v.at[sl]],
                                       yrows_v.at[sl], sem_y))

    @pl.when(wid == 0)
    def _():
        pltpu.sync_copy(neg_hbm, negidx_v)
        pltpu.async_copy(wo_hbm.at[negidx_v], negrows_v, sem_n).wait()
        pltpu.sync_copy(negrows_v, samp_out)

    for c in copies:
        c.wait()

    pltpu.sync_copy(xrows_v, vi_out.at[pl.ds(base, ROWS_PER_W)])
    pltpu.sync_copy(yrows_v, vo_out.at[pl.ds(base, ROWS_PER_W)])


@functools.cache
def _build_sc_gather_call():
    return functools.partial(
        pl.kernel,
        out_type=(
            jax.ShapeDtypeStruct((B, D), jnp.float32),
            jax.ShapeDtypeStruct((B, D), jnp.float32),
            jax.ShapeDtypeStruct((NEG_PAD, D), jnp.float32),
        ),
        mesh=plsc.VectorSubcoreMesh(
            core_axis_name="c", subcore_axis_name="s",
            num_cores=NUM_SC, num_subcores=NUM_SUBCORES),
        compiler_params=pltpu.CompilerParams(use_tc_tiling_on_sc=False),
        scratch_types=(
            pltpu.VMEM((ROWS_PER_W,), jnp.int32),
            pltpu.VMEM((ROWS_PER_W,), jnp.int32),
            pltpu.VMEM((NEG_PAD,), jnp.int32),
            pltpu.VMEM((ROWS_PER_W, D), jnp.float32),
            pltpu.VMEM((ROWS_PER_W, D), jnp.float32),
            pltpu.VMEM((NEG_PAD, D), jnp.float32),
            pltpu.SemaphoreType.DMA,
            pltpu.SemaphoreType.DMA,
            pltpu.SemaphoreType.DMA,
        ),
    )(_sc_gather)


BLK = 512
GRID = B // BLK


def _log_sigmoid(z):
    return jnp.minimum(z, 0.0) - jnp.log(1.0 + jnp.exp(-jnp.abs(z)))


def _tc_loss(vi_ref, vo_ref, samp_ref, out_ref):
    i = pl.program_id(0)
    vi = vi_ref[...]
    vo = vo_ref[...]
    pos = jnp.sum(vi * vo, axis=1)
    negs = lax.dot_general(vi, samp_ref[...],
                           (((1,), (1,)), ((), ())),
                           preferred_element_type=jnp.float32)
    pos_l = _log_sigmoid(pos)
    col = lax.broadcasted_iota(jnp.int32, (BLK, NEG_PAD), 1)
    neg_l = jnp.sum(jnp.where(col < NEG, _log_sigmoid(-negs), 0.0), axis=1)
    part = jnp.sum(pos_l + neg_l).reshape(1, 1)

    @pl.when(i == 0)
    def _():
        out_ref[...] = jnp.zeros((1, 1), jnp.float32)

    out_ref[...] += part

    @pl.when(i == GRID - 1)
    def _():
        out_ref[...] = out_ref[...] * (-1.0 / B)


@functools.cache
def _build_tc_loss_call(interpret=False):
    return pl.pallas_call(
        _tc_loss,
        grid=(GRID,),
        in_specs=[
            pl.BlockSpec((BLK, D), lambda i: (i, 0)),
            pl.BlockSpec((BLK, D), lambda i: (i, 0)),
            pl.BlockSpec((NEG_PAD, D), lambda i: (0, 0)),
        ],
        out_specs=pl.BlockSpec((1, 1), lambda i: (0, 0)),
        out_shape=jax.ShapeDtypeStruct((1, 1), jnp.float32),
        interpret=interpret,
    )


def kernel(x_lookup, y_lookup, neg_lookup, WI, WO):
    neg_pad = jnp.concatenate(
        [neg_lookup.astype(jnp.int32),
         jnp.zeros((NEG_PAD - NEG,), jnp.int32)])
    vi, vo, samp = _build_sc_gather_call()(
        x_lookup.astype(jnp.int32), y_lookup.astype(jnp.int32), neg_pad,
        WI, WO)
    loss = _build_tc_loss_call()(vi, vo, samp)
    return loss[0, 0]


# speedup vs baseline: 2.3125x; 2.3125x over previous
import functools

import jax
import jax.numpy as jnp
from jax import lax
from jax.experimental import pallas as pl
from jax.experimental.pallas import tpu as pltpu
from jax.experimental.pallas import tpu_sc as plsc

B = 16384
D = 64
NEG = 5
NEG_PAD = 16

NUM_SC = 2
NUM_SUBCORES = 16
ROWS_PER_TILE = B // NUM_SUBCORES
NB = 4
NGROUPS = ROWS_PER_TILE // NB
HALF = ROWS_PER_TILE // 2
NITER = NGROUPS // 4


def _fetch_row(tbl, v, fbuf, slot, sem):
    off = pl.multiple_of((v >> 7) * 128, 128)
    return pltpu.async_copy(tbl.at[:, pl.ds(off, 128)], fbuf.at[slot], sem)


def _extract_row(v, fbuf, slot, voutbuf, rloc):
    lane_v = jnp.full((16,), v & 127, jnp.int32)
    row_v = jnp.full((16,), rloc >> 1, jnp.int32)
    colbase = (rloc & 1) * 64
    iota = lax.iota(jnp.int32, 16)
    for k in range(4):
        vec = plsc.load_gather(fbuf.at[slot], [iota + (16 * k), lane_v])
        plsc.store_scatter(voutbuf, [row_v, iota + (colbase + 16 * k)], vec)


def _drain_group(tbl, fbuf, q, sem):
    for j in range(NB):
        pltpu.make_async_copy(tbl.at[:, pl.ds(0, 128)],
                              fbuf.at[q * NB + j], sem).wait()


def _sc_body(x_hbm, y_hbm, neg_hbm, wit_hbm, wot_hbm,
             vi_out, vo_out, samp_out,
             idx_v, negidx_v, fbuf, voutbuf, sampbuf,
             sem0, sem1, semn):
    c = lax.axis_index("c")
    s = lax.axis_index("s")
    base = s * ROWS_PER_TILE
    sems = (sem0, sem1)

    def run_table(idx_hbm, tbl, out):
        pltpu.sync_copy(idx_hbm.at[pl.ds(base, ROWS_PER_TILE)], idx_v)

        def step(i, carry):
            voff = pl.multiple_of(16 * i, 16)
            vec = idx_v[pl.ds(voff, 16)]

            for gg in range(4):
                p = gg % 2

                for j in range(NB):
                    _fetch_row(tbl, vec[gg * NB + j], fbuf,
                               p * NB + j, sems[p])

                if gg == 1:
                    @pl.when(i == NITER // 2)
                    def _():
                        off0 = pl.multiple_of(s * (ROWS_PER_TILE // 2), 256)
                        pltpu.sync_copy(voutbuf,
                                        out.at[pl.ds(off0, HALF // 2)])

                q = 1 - p
                if gg == 0:
                    @pl.when(i > 0)
                    def _():
                        pvec = idx_v[pl.ds(pl.multiple_of(16 * i - 16, 16),
                                           16)]
                        _drain_group(tbl, fbuf, q, sems[q])
                        for j in range(NB):
                            r = (4 * i - 1) * NB + j
                            _extract_row(pvec[12 + j], fbuf, q * NB + j,
                                         voutbuf, r % HALF)
                else:
                    _drain_group(tbl, fbuf, q, sems[q])
                    for j in range(NB):
                        r = (4 * i + gg - 1) * NB + j
                        _extract_row(vec[(gg - 1) * NB + j], fbuf,
                                     q * NB + j, voutbuf, r % HALF)
            return carry

        lax.fori_loop(0, NITER, step, 0)

        lvec = idx_v[pl.ds(16 * (NITER - 1), 16)]
        _drain_group(tbl, fbuf, 1, sems[1])
        for j in range(NB):
            r = (NGROUPS - 1) * NB + j
            _extract_row(lvec[12 + j], fbuf, NB + j, voutbuf, r % HALF)
        off1 = pl.multiple_of(s * (ROWS_PER_TILE // 2) + HALF // 2, 256)
        pltpu.sync_copy(voutbuf, out.at[pl.ds(off1, HALF // 2)])

    @pl.when(c == 0)
    def _():
        run_table(x_hbm, wit_hbm, vi_out)

    @pl.when(c == 1)
    def _():
        run_table(y_hbm, wot_hbm, vo_out)

    @pl.when(jnp.logical_and(c == 1, s == 0))
    def _():
        pltpu.sync_copy(neg_hbm, negidx_v)
        nvec = negidx_v[pl.ds(0, 16)]
        for h in range(2):
            cps = [_fetch_row(wot_hbm, nvec[h * 8 + j], fbuf, j, semn)
                   for j in range(8)]
            for cp in cps:
                cp.wait()
            for j in range(8):
                _extract_row(nvec[h * 8 + j], fbuf, j, sampbuf, h * 8 + j)
        pltpu.sync_copy(sampbuf, samp_out)


@functools.cache
def _build_sc_gather_call():
    return functools.partial(
        pl.kernel,
        out_type=(
            jax.ShapeDtypeStruct((B // 2, 128), jnp.float32),
            jax.ShapeDtypeStruct((B // 2, 128), jnp.float32),
            jax.ShapeDtypeStruct((NEG_PAD // 2, 128), jnp.float32),
        ),
        mesh=plsc.VectorSubcoreMesh(
            core_axis_name="c", subcore_axis_name="s",
            num_cores=NUM_SC, num_subcores=NUM_SUBCORES),
        compiler_params=pltpu.CompilerParams(needs_layout_passes=False),
        scratch_types=(
            pltpu.VMEM((ROWS_PER_TILE,), jnp.int32),
            pltpu.VMEM((NEG_PAD,), jnp.int32),
            pltpu.VMEM((2 * NB, D, 128), jnp.float32),
            pltpu.VMEM((HALF // 2, 128), jnp.float32),
            pltpu.VMEM((NEG_PAD // 2, 128), jnp.float32),
            pltpu.SemaphoreType.DMA,
            pltpu.SemaphoreType.DMA,
            pltpu.SemaphoreType.DMA,
        ),
    )(_sc_body)


BLK2 = 256
GRID = (B // 2) // BLK2


def _log_sigmoid(z):
    return jnp.minimum(z, 0.0) - jnp.log(1.0 + jnp.exp(-jnp.abs(z)))


def _tc_loss(vi_ref, vo_ref, samp_ref, out_ref):
    i = pl.program_id(0)
    vi2 = vi_ref[...]
    vo2 = vo_ref[...]
    prod = vi2 * vo2
    pos_e = jnp.sum(prod[:, :D], axis=1)
    pos_o = jnp.sum(prod[:, D:], axis=1)
    samp2 = samp_ref[...]
    sampcat = jnp.concatenate([samp2[:, :D], samp2[:, D:]], axis=0)
    dn = (((1,), (1,)), ((), ()))
    negs_e = lax.dot_general(vi2[:, :D], sampcat, dn,
                             preferred_element_type=jnp.float32)
    negs_o = lax.dot_general(vi2[:, D:], sampcat, dn,
                             preferred_element_type=jnp.float32)
    col = lax.broadcasted_iota(jnp.int32, (BLK2, NEG_PAD), 1)
    valid = (col < 3) | (col == 8) | (col == 9)
    neg_l = (jnp.sum(jnp.where(valid, _log_sigmoid(-negs_e), 0.0), axis=1)
             + jnp.sum(jnp.where(valid, _log_sigmoid(-negs_o), 0.0), axis=1))
    part = jnp.sum(_log_sigmoid(pos_e) + _log_sigmoid(pos_o)
                   + neg_l).reshape(1, 1)

    @pl.when(i == 0)
    def _():
        out_ref[...] = jnp.zeros((1, 1), jnp.float32)

    out_ref[...] += part

    @pl.when(i == GRID - 1)
    def _():
        out_ref[...] = out_ref[...] * (-1.0 / B)


@functools.cache
def _build_tc_loss_call(interpret=False):
    return pl.pallas_call(
        _tc_loss,
        grid=(GRID,),
        in_specs=[
            pl.BlockSpec((BLK2, 128), lambda i: (i, 0)),
            pl.BlockSpec((BLK2, 128), lambda i: (i, 0)),
            pl.BlockSpec((NEG_PAD // 2, 128), lambda i: (0, 0)),
        ],
        out_specs=pl.BlockSpec((1, 1), lambda i: (0, 0)),
        out_shape=jax.ShapeDtypeStruct((1, 1), jnp.float32),
        interpret=interpret,
    )


def kernel(x_lookup, y_lookup, neg_lookup, WI, WO):
    neg_pad = jnp.concatenate(
        [neg_lookup.astype(jnp.int32),
         jnp.zeros((NEG_PAD - NEG,), jnp.int32)])
    vi, vo, samp = _build_sc_gather_call()(
        x_lookup.astype(jnp.int32), y_lookup.astype(jnp.int32), neg_pad,
        WI.T, WO.T)
    loss = _build_tc_loss_call()(vi, vo, samp)
    return loss[0, 0]
